# baseline (speedup 1.0000x reference)
import functools

import jax
import jax.numpy as jnp
from jax import lax
from jax.experimental import pallas as pl
from jax.experimental.pallas import tpu as pltpu
from jax.experimental.pallas import tpu_sc as plsc

_REASON_TOKEN_ID = 1000
_B, _T = 4, 4096
_L = 16
_NC = 2


@functools.partial(
    pl.kernel,
    mesh=plsc.VectorSubcoreMesh(core_axis_name="c", subcore_axis_name="s"),
    out_type=jax.ShapeDtypeStruct((_B, _L), jnp.int32),
    scratch_types=[
        pltpu.VMEM((_T,), jnp.int32),
        pltpu.VMEM((_L,), jnp.int32),
    ],
)
def _sc_route_mask(ids_hbm, out_hbm, row_v, flag_v):
    wid = lax.axis_index("s") * _NC + lax.axis_index("c")

    @pl.when(wid < _B)
    def _():
        pltpu.sync_copy(ids_hbm.at[wid], row_v)

        _UNROLL = 4

        def body(i, acc):
            base = i * (_UNROLL * _L)
            for k in range(_UNROLL):
                v = row_v[pl.ds(base + k * _L, _L)]
                hit = jnp.where(v == _REASON_TOKEN_ID,
                                jnp.full((_L,), 1, jnp.int32),
                                jnp.full((_L,), 0, jnp.int32))
                acc = acc | hit
            return acc

        acc = lax.fori_loop(0, _T // (_UNROLL * _L), body,
                            jnp.full((_L,), 0, jnp.int32))
        for shift in (1, 2, 4, 8):
            perm = (lax.iota(jnp.int32, _L) + shift) & (_L - 1)
            acc = acc | acc.at[perm].get(mode="promise_in_bounds")
        flag_v[...] = acc
        pltpu.sync_copy(flag_v, out_hbm.at[wid])


_NDMA = 8


def _copy_body(hs_ref, out_ref, sems):
    rows = hs_ref.shape[0]
    chunk = rows // _NDMA
    copies = [
        pltpu.make_async_copy(
            hs_ref.at[pl.ds(k * chunk, chunk)],
            out_ref.at[pl.ds(k * chunk, chunk)],
            sems.at[k],
        )
        for k in range(_NDMA)
    ]
    for c in copies:
        c.start()
    for c in copies:
        c.wait()


def _copy_hbm(hs2d):
    return pl.pallas_call(
        _copy_body,
        in_specs=[pl.BlockSpec(memory_space=pltpu.MemorySpace.HBM)],
        out_specs=pl.BlockSpec(memory_space=pltpu.MemorySpace.HBM),
        out_shape=jax.ShapeDtypeStruct(hs2d.shape, hs2d.dtype),
        scratch_shapes=[pltpu.SemaphoreType.DMA((_NDMA,))],
    )(hs2d)


def kernel(input_ids, hidden_states):
    flags = _sc_route_mask(input_ids.astype(jnp.int32))
    route_mask = flags[:, 0] > 0
    b, t, d = hidden_states.shape
    out = _copy_hbm(hidden_states.reshape(b * t, d)).reshape(b, t, d)
    return (out, route_mask)


# speedup vs baseline: 39.9444x; 39.9444x over previous
import functools

import jax
import jax.numpy as jnp
from jax import lax
from jax.experimental import pallas as pl
from jax.experimental.pallas import tpu as pltpu
from jax.experimental.pallas import tpu_sc as plsc

_REASON_TOKEN_ID = 1000
_B, _T = 4, 4096
_L = 16
_NC = 2


@functools.partial(
    pl.kernel,
    mesh=plsc.VectorSubcoreMesh(core_axis_name="c", subcore_axis_name="s"),
    out_type=jax.ShapeDtypeStruct((_B, _L), jnp.int32),
    scratch_types=[
        pltpu.VMEM((_T,), jnp.int32),
        pltpu.VMEM((_L,), jnp.int32),
    ],
)
def _sc_route_mask(ids_hbm, out_hbm, row_v, flag_v):
    wid = lax.axis_index("s") * _NC + lax.axis_index("c")

    @pl.when(wid < _B)
    def _():
        pltpu.sync_copy(ids_hbm.at[wid], row_v)

        _UNROLL = 4

        def body(i, acc):
            base = i * (_UNROLL * _L)
            for k in range(_UNROLL):
                v = row_v[pl.ds(base + k * _L, _L)]
                hit = jnp.where(v == _REASON_TOKEN_ID,
                                jnp.full((_L,), 1, jnp.int32),
                                jnp.full((_L,), 0, jnp.int32))
                acc = acc | hit
            return acc

        acc = lax.fori_loop(0, _T // (_UNROLL * _L), body,
                            jnp.full((_L,), 0, jnp.int32))
        for shift in (1, 2, 4, 8):
            perm = (lax.iota(jnp.int32, _L) + shift) & (_L - 1)
            acc = acc | acc.at[perm].get(mode="promise_in_bounds")
        flag_v[...] = acc
        pltpu.sync_copy(flag_v, out_hbm.at[wid])


_BLOCK_ROWS = 512


def _copy_block(hs_ref, out_ref):
    out_ref[...] = hs_ref[...]


def _copy_hbm(hs2d):
    rows, cols = hs2d.shape
    return pl.pallas_call(
        _copy_block,
        grid=(rows // _BLOCK_ROWS,),
        in_specs=[pl.BlockSpec((_BLOCK_ROWS, cols), lambda i: (i, 0))],
        out_specs=pl.BlockSpec((_BLOCK_ROWS, cols), lambda i: (i, 0)),
        out_shape=jax.ShapeDtypeStruct(hs2d.shape, hs2d.dtype),
    )(hs2d)


def kernel(input_ids, hidden_states):
    flags = _sc_route_mask(input_ids.astype(jnp.int32))
    route_mask = flags[:, 0] > 0
    b, t, d = hidden_states.shape
    out = _copy_hbm(hidden_states.reshape(b * t, d)).reshape(b, t, d)
    return (out, route_mask)


# speedup vs baseline: 40.5799x; 1.0159x over previous
import functools

import jax
import jax.numpy as jnp
from jax import lax
from jax.experimental import pallas as pl
from jax.experimental.pallas import tpu as pltpu
from jax.experimental.pallas import tpu_sc as plsc

_REASON_TOKEN_ID = 1000
_B, _T = 4, 4096
_L = 16
_NC = 2


@functools.partial(
    pl.kernel,
    mesh=plsc.VectorSubcoreMesh(core_axis_name="c", subcore_axis_name="s"),
    out_type=jax.ShapeDtypeStruct((_B, _L), jnp.int32),
    scratch_types=[
        pltpu.VMEM((_T,), jnp.int32),
        pltpu.VMEM((_L,), jnp.int32),
    ],
)
def _sc_route_mask(ids_hbm, out_hbm, row_v, flag_v):
    wid = lax.axis_index("s") * _NC + lax.axis_index("c")

    @pl.when(wid < _B)
    def _():
        pltpu.sync_copy(ids_hbm.at[wid], row_v)

        _UNROLL = 4

        def body(i, acc):
            base = i * (_UNROLL * _L)
            for k in range(_UNROLL):
                v = row_v[pl.ds(base + k * _L, _L)]
                hit = jnp.where(v == _REASON_TOKEN_ID,
                                jnp.full((_L,), 1, jnp.int32),
                                jnp.full((_L,), 0, jnp.int32))
                acc = acc | hit
            return acc

        acc = lax.fori_loop(0, _T // (_UNROLL * _L), body,
                            jnp.full((_L,), 0, jnp.int32))
        for shift in (1, 2, 4, 8):
            perm = (lax.iota(jnp.int32, _L) + shift) & (_L - 1)
            acc = acc | acc.at[perm].get(mode="promise_in_bounds")
        flag_v[...] = acc
        pltpu.sync_copy(flag_v, out_hbm.at[wid])


_BLOCK_ROWS = 1024


def _copy_block(hs_ref, out_ref):
    out_ref[...] = hs_ref[...]


def _copy_hbm(hs2d):
    rows, cols = hs2d.shape
    return pl.pallas_call(
        _copy_block,
        grid=(rows // _BLOCK_ROWS,),
        in_specs=[pl.BlockSpec((_BLOCK_ROWS, cols), lambda i: (i, 0))],
        out_specs=pl.BlockSpec((_BLOCK_ROWS, cols), lambda i: (i, 0)),
        out_shape=jax.ShapeDtypeStruct(hs2d.shape, hs2d.dtype),
    )(hs2d)


def kernel(input_ids, hidden_states):
    b, t, d = hidden_states.shape
    out = _copy_hbm(hidden_states.reshape(b * t, d)).reshape(b, t, d)
    flags = _sc_route_mask(input_ids.astype(jnp.int32))
    route_mask = flags[:, 0] > 0
    return (out, route_mask)


# speedup vs baseline: 41.1158x; 1.0132x over previous
import functools

import jax
import jax.numpy as jnp
from jax import lax
from jax.experimental import pallas as pl
from jax.experimental.pallas import tpu as pltpu
from jax.experimental.pallas import tpu_sc as plsc

_REASON_TOKEN_ID = 1000
_B, _T = 4, 4096
_L = 16
_NC = 2
_MESH_CORES = 1


@functools.partial(
    pl.kernel,
    mesh=plsc.VectorSubcoreMesh(core_axis_name="c", subcore_axis_name="s",
                                num_cores=_MESH_CORES),
    out_type=jax.ShapeDtypeStruct((_B, _L), jnp.int32),
    scratch_types=[
        pltpu.VMEM((_T,), jnp.int32),
        pltpu.VMEM((_L,), jnp.int32),
    ],
)
def _sc_route_mask(ids_hbm, out_hbm, row_v, flag_v):
    wid = lax.axis_index("s") * _MESH_CORES + lax.axis_index("c")

    @pl.when(wid < _B)
    def _():
        pltpu.sync_copy(ids_hbm.at[wid], row_v)

        _UNROLL = 4

        def body(i, acc):
            base = i * (_UNROLL * _L)
            for k in range(_UNROLL):
                v = row_v[pl.ds(base + k * _L, _L)]
                hit = jnp.where(v == _REASON_TOKEN_ID,
                                jnp.full((_L,), 1, jnp.int32),
                                jnp.full((_L,), 0, jnp.int32))
                acc = acc | hit
            return acc

        acc = lax.fori_loop(0, _T // (_UNROLL * _L), body,
                            jnp.full((_L,), 0, jnp.int32))
        for shift in (1, 2, 4, 8):
            perm = (lax.iota(jnp.int32, _L) + shift) & (_L - 1)
            acc = acc | acc.at[perm].get(mode="promise_in_bounds")
        flag_v[...] = acc
        pltpu.sync_copy(flag_v, out_hbm.at[wid])


_BLOCK_ROWS = 1024


def _copy_block(hs_ref, out_ref):
    out_ref[...] = hs_ref[...]


def _copy_hbm(hs2d):
    rows, cols = hs2d.shape
    return pl.pallas_call(
        _copy_block,
        grid=(rows // _BLOCK_ROWS,),
        in_specs=[pl.BlockSpec((_BLOCK_ROWS, cols), lambda i: (i, 0))],
        out_specs=pl.BlockSpec((_BLOCK_ROWS, cols), lambda i: (i, 0)),
        out_shape=jax.ShapeDtypeStruct(hs2d.shape, hs2d.dtype),
    )(hs2d)


def kernel(input_ids, hidden_states):
    b, t, d = hidden_states.shape
    out = _copy_hbm(hidden_states.reshape(b * t, d)).reshape(b, t, d)
    flags = _sc_route_mask(input_ids.astype(jnp.int32))
    route_mask = flags[:, 0] > 0
    return (out, route_mask)


# speedup vs baseline: 47.8390x; 1.1635x over previous
import functools

import jax
import jax.numpy as jnp
from jax import lax
from jax.experimental import pallas as pl
from jax.experimental.pallas import tpu as pltpu
from jax.experimental.pallas import tpu_sc as plsc

_REASON_TOKEN_ID = 1000
_B, _T = 4, 4096
_L = 16
_NC = 2
_MESH_CORES = 1


@functools.partial(
    pl.kernel,
    mesh=plsc.VectorSubcoreMesh(core_axis_name="c", subcore_axis_name="s",
                                num_cores=_MESH_CORES),
    out_type=jax.ShapeDtypeStruct((_B, _L), jnp.int32),
    scratch_types=[
        pltpu.VMEM((_T,), jnp.int32),
        pltpu.VMEM((_L,), jnp.int32),
    ],
)
def _sc_route_mask(ids_hbm, out_hbm, row_v, flag_v):
    wid = lax.axis_index("s") * _MESH_CORES + lax.axis_index("c")

    @pl.when(wid < _B)
    def _():
        pltpu.sync_copy(ids_hbm.at[wid], row_v)

        _UNROLL = 4

        def body(i, acc):
            base = i * (_UNROLL * _L)
            for k in range(_UNROLL):
                v = row_v[pl.ds(base + k * _L, _L)]
                hit = jnp.where(v == _REASON_TOKEN_ID,
                                jnp.full((_L,), 1, jnp.int32),
                                jnp.full((_L,), 0, jnp.int32))
                acc = acc | hit
            return acc

        acc = lax.fori_loop(0, _T // (_UNROLL * _L), body,
                            jnp.full((_L,), 0, jnp.int32))
        for shift in (1, 2, 4, 8):
            perm = (lax.iota(jnp.int32, _L) + shift) & (_L - 1)
            acc = acc | acc.at[perm].get(mode="promise_in_bounds")
        flag_v[...] = acc
        pltpu.sync_copy(flag_v, out_hbm.at[wid])


_BLOCK_ROWS = 1024


def _copy_mask_block(ids_ref, hs_ref, out_ref, mask_ref):
    out_ref[...] = hs_ref[...]

    @pl.when(pl.program_id(0) == 0)
    def _():
        hit = (ids_ref[...] == _REASON_TOKEN_ID).astype(jnp.int32)
        mask_ref[...] = jnp.max(hit, axis=-1, keepdims=True)


def _copy_and_mask(ids, hs2d):
    rows, cols = hs2d.shape
    return pl.pallas_call(
        _copy_mask_block,
        grid=(rows // _BLOCK_ROWS,),
        in_specs=[
            pl.BlockSpec((_B, _T), lambda i: (0, 0)),
            pl.BlockSpec((_BLOCK_ROWS, cols), lambda i: (i, 0)),
        ],
        out_specs=[
            pl.BlockSpec((_BLOCK_ROWS, cols), lambda i: (i, 0)),
            pl.BlockSpec((_B, 1), lambda i: (0, 0)),
        ],
        out_shape=[
            jax.ShapeDtypeStruct(hs2d.shape, hs2d.dtype),
            jax.ShapeDtypeStruct((_B, 1), jnp.int32),
        ],
    )(ids, hs2d)


def kernel(input_ids, hidden_states):
    b, t, d = hidden_states.shape
    out, flags = _copy_and_mask(input_ids.astype(jnp.int32),
                                hidden_states.reshape(b * t, d))
    route_mask = flags[:, 0] > 0
    return (out.reshape(b, t, d), route_mask)
